# Initial kernel scaffold; baseline (speedup 1.0000x reference)
import jax
import jax.numpy as jnp
from jax.experimental import pallas as pl


def kernel(x, edge_index, edge_type, w, b, edge_embeddings):
    raise NotImplementedError("write your pallas kernel here")


# speedup vs baseline: 146.6470x; 146.6470x over previous
import functools

import jax
import jax.numpy as jnp
from jax import lax
from jax.experimental import pallas as pl
from jax.experimental.pallas import tpu as pltpu
from jax.experimental.pallas import tpu_sc as plsc

N = 10000
E = 320000
D = 128
NPAD = 10240
PADBIN = N

NC = 2
NS = 16
CW = 128
CPK = 80
EPADT = CPK * CW
BPT = NPAD // NS


def _sc_hist_body(idx_hbm, out_hbm, idx_v, ones_v, zero_v, obuf_v,
                  hist_src, hist_dst):
    c = lax.axis_index("c")
    s = lax.axis_index("s")

    @pl.loop(0, 8)
    def _(i):
        ones_v[pl.ds(i * 16, 16)] = jnp.ones((16,), jnp.float32)

    @pl.loop(0, BPT // 16)
    def _(i):
        zero_v[pl.ds(i * 16, 16)] = jnp.zeros((16,), jnp.float32)

    pltpu.sync_copy(zero_v, hist_src.at[pl.ds(s * BPT, BPT)])
    pltpu.sync_copy(zero_v, hist_dst.at[pl.ds(s * BPT, BPT)])
    plsc.subcore_barrier()

    pltpu.sync_copy(idx_hbm.at[c, s], idx_v)

    @pl.loop(0, CPK)
    def _(j):
        pltpu.sync_copy(ones_v, hist_src.at[idx_v.at[j]], add=True)

    @pl.loop(CPK, 2 * CPK)
    def _(j):
        pltpu.sync_copy(ones_v, hist_dst.at[idx_v.at[j]], add=True)

    plsc.subcore_barrier()

    pltpu.sync_copy(hist_src.at[pl.ds(s * BPT, BPT)], obuf_v)
    pltpu.sync_copy(obuf_v, out_hbm.at[c, 0, pl.ds(s * BPT, BPT)])
    pltpu.sync_copy(hist_dst.at[pl.ds(s * BPT, BPT)], obuf_v)
    pltpu.sync_copy(obuf_v, out_hbm.at[c, 1, pl.ds(s * BPT, BPT)])


_sc_hist = pl.kernel(
    _sc_hist_body,
    out_type=jax.ShapeDtypeStruct((NC, 2, NPAD), jnp.float32),
    mesh=plsc.VectorSubcoreMesh(core_axis_name="c", subcore_axis_name="s",
                                num_cores=NC, num_subcores=NS),
    scratch_types=[
        pltpu.VMEM((2 * CPK, CW), jnp.int32),
        pltpu.VMEM((CW,), jnp.float32),
        pltpu.VMEM((BPT,), jnp.float32),
        pltpu.VMEM((BPT,), jnp.float32),
        pltpu.VMEM_SHARED((NPAD,), jnp.float32),
        pltpu.VMEM_SHARED((NPAD,), jnp.float32),
    ],
)


def _tc_scale_body(x_ref, p_ref, o_ref):
    p = p_ref[...]
    out_deg = p[0, 0] + p[1, 0]
    in_deg = p[0, 1] + p[1, 1]
    div = jnp.where(in_deg > 0.0, jnp.maximum(out_deg, 1.0), 1.0)
    recip = (1.0 / div)[:N]
    o_ref[...] = x_ref[...] * recip[:, None]


@jax.jit
def kernel(x, edge_index, edge_type, w, b, edge_embeddings):
    del edge_type, w, b, edge_embeddings
    src = edge_index[0].astype(jnp.int32)
    dst = edge_index[1].astype(jnp.int32)
    pad = jnp.full((NC * NS * EPADT - E,), PADBIN, jnp.int32)
    srcp = jnp.concatenate([src, pad]).reshape(NC, NS, CPK, CW)
    dstp = jnp.concatenate([dst, pad]).reshape(NC, NS, CPK, CW)
    idx = jnp.concatenate([srcp, dstp], axis=2)

    partial = _sc_hist(idx)

    return pl.pallas_call(
        _tc_scale_body,
        out_shape=jax.ShapeDtypeStruct((N, D), jnp.float32),
    )(x, partial)


# speedup vs baseline: 165.6041x; 1.1293x over previous
import jax
import jax.numpy as jnp
from jax import lax
from jax.experimental import pallas as pl
from jax.experimental.pallas import tpu as pltpu
from jax.experimental.pallas import tpu_sc as plsc

N = 10000
E = 320000
D = 128
NPAD = 10240

NC = 2
NS = 16
CW = 80
CPT = 125
BPT = NPAD // NS


def _sc_hist_body(idx_hbm, out_hbm, idx_v, ones_v, zero_v, obuf_v,
                  hist_src, hist_dst):
    c = lax.axis_index("c")
    s = lax.axis_index("s")
    wid = c * NS + s

    @pl.loop(0, CW // 16)
    def _(i):
        ones_v[pl.ds(i * 16, 16)] = jnp.ones((16,), jnp.float32)

    @pl.loop(0, BPT // 16)
    def _(i):
        zero_v[pl.ds(i * 16, 16)] = jnp.zeros((16,), jnp.float32)

    pltpu.sync_copy(zero_v, hist_src.at[pl.ds(s * BPT, BPT)])
    pltpu.sync_copy(zero_v, hist_dst.at[pl.ds(s * BPT, BPT)])
    plsc.subcore_barrier()

    pltpu.sync_copy(idx_hbm.at[0, wid], idx_v)

    @pl.loop(0, CPT)
    def _(j):
        pltpu.sync_copy(ones_v, hist_src.at[idx_v.at[j]], add=True)

    pltpu.sync_copy(idx_hbm.at[1, wid], idx_v)

    @pl.loop(0, CPT)
    def _(j):
        pltpu.sync_copy(ones_v, hist_dst.at[idx_v.at[j]], add=True)

    plsc.subcore_barrier()

    pltpu.sync_copy(hist_src.at[pl.ds(s * BPT, BPT)], obuf_v)
    pltpu.sync_copy(obuf_v, out_hbm.at[c, 0, pl.ds(s * BPT, BPT)])
    pltpu.sync_copy(hist_dst.at[pl.ds(s * BPT, BPT)], obuf_v)
    pltpu.sync_copy(obuf_v, out_hbm.at[c, 1, pl.ds(s * BPT, BPT)])


_sc_hist = pl.kernel(
    _sc_hist_body,
    out_type=jax.ShapeDtypeStruct((NC, 2, NPAD), jnp.float32),
    mesh=plsc.VectorSubcoreMesh(core_axis_name="c", subcore_axis_name="s",
                                num_cores=NC, num_subcores=NS),
    scratch_types=[
        pltpu.VMEM((CPT, CW), jnp.int32),
        pltpu.VMEM((CW,), jnp.float32),
        pltpu.VMEM((BPT,), jnp.float32),
        pltpu.VMEM((BPT,), jnp.float32),
        pltpu.VMEM_SHARED((NPAD,), jnp.float32),
        pltpu.VMEM_SHARED((NPAD,), jnp.float32),
    ],
)


def _tc_scale_body(x_ref, p_ref, o_ref):
    p = p_ref[...]
    out_deg = p[0, 0] + p[1, 0]
    in_deg = p[0, 1] + p[1, 1]
    div = jnp.where(in_deg > 0.0, jnp.maximum(out_deg, 1.0), 1.0)
    recip = (1.0 / div)[:N]
    o_ref[...] = x_ref[...] * recip[:, None]


@jax.jit
def kernel(x, edge_index, edge_type, w, b, edge_embeddings):
    del edge_type, w, b, edge_embeddings
    idx = edge_index.astype(jnp.int32).reshape(2, NC * NS, CPT, CW)

    partial = _sc_hist(idx)

    return pl.pallas_call(
        _tc_scale_body,
        out_shape=jax.ShapeDtypeStruct((N, D), jnp.float32),
    )(x, partial)


# speedup vs baseline: 194.0764x; 1.1719x over previous
import jax
import jax.numpy as jnp
from jax import lax
from jax.experimental import pallas as pl
from jax.experimental.pallas import tpu as pltpu
from jax.experimental.pallas import tpu_sc as plsc

N = 10000
E = 320000
D = 128
NPAD = 10240

NC = 2
NS = 16
EPT = E // (NC * NS)
BPT = NPAD // NS


def _sc_hist_body(src_hbm, dst_hbm, out_hbm, iv_s, iv_d, ones_v, zero_v,
                  obuf_v, sem, hist_src, hist_dst):
    c = lax.axis_index("c")
    s = lax.axis_index("s")
    base = (c * NS + s) * EPT

    load_s = pltpu.async_copy(src_hbm.at[pl.ds(base, EPT)], iv_s, sem)
    load_d = pltpu.async_copy(dst_hbm.at[pl.ds(base, EPT)], iv_d, sem)

    @pl.loop(0, EPT // 16)
    def _(i):
        ones_v[pl.ds(i * 16, 16)] = jnp.ones((16,), jnp.float32)

    @pl.loop(0, BPT // 16)
    def _(i):
        zero_v[pl.ds(i * 16, 16)] = jnp.zeros((16,), jnp.float32)

    pltpu.sync_copy(zero_v, hist_src.at[pl.ds(s * BPT, BPT)])
    pltpu.sync_copy(zero_v, hist_dst.at[pl.ds(s * BPT, BPT)])
    plsc.subcore_barrier()

    load_s.wait()
    pltpu.sync_copy(ones_v, hist_src.at[iv_s], add=True)
    load_d.wait()
    pltpu.sync_copy(ones_v, hist_dst.at[iv_d], add=True)

    plsc.subcore_barrier()

    pltpu.sync_copy(hist_src.at[pl.ds(s * BPT, BPT)], obuf_v)
    pltpu.sync_copy(obuf_v, out_hbm.at[c, 0, pl.ds(s * BPT, BPT)])
    pltpu.sync_copy(hist_dst.at[pl.ds(s * BPT, BPT)], obuf_v)
    pltpu.sync_copy(obuf_v, out_hbm.at[c, 1, pl.ds(s * BPT, BPT)])


_sc_hist = pl.kernel(
    _sc_hist_body,
    out_type=jax.ShapeDtypeStruct((NC, 2, NPAD), jnp.float32),
    mesh=plsc.VectorSubcoreMesh(core_axis_name="c", subcore_axis_name="s",
                                num_cores=NC, num_subcores=NS),
    scratch_types=[
        pltpu.VMEM((EPT,), jnp.int32),
        pltpu.VMEM((EPT,), jnp.int32),
        pltpu.VMEM((EPT,), jnp.float32),
        pltpu.VMEM((BPT,), jnp.float32),
        pltpu.VMEM((BPT,), jnp.float32),
        pltpu.SemaphoreType.DMA,
        pltpu.VMEM_SHARED((NPAD,), jnp.float32),
        pltpu.VMEM_SHARED((NPAD,), jnp.float32),
    ],
)


def _tc_scale_body(x_ref, p_ref, o_ref):
    p = p_ref[...]
    out_deg = p[0, 0] + p[1, 0]
    in_deg = p[0, 1] + p[1, 1]
    div = jnp.where(in_deg > 0.0, jnp.maximum(out_deg, 1.0), 1.0)
    recip = (1.0 / div)[:N]
    o_ref[...] = x_ref[...] * recip[:, None]


@jax.jit
def kernel(x, edge_index, edge_type, w, b, edge_embeddings):
    del edge_type, w, b, edge_embeddings
    ei = edge_index.astype(jnp.int32)
    src = ei[0]
    dst = ei[1]

    partial = _sc_hist(src, dst)

    return pl.pallas_call(
        _tc_scale_body,
        out_shape=jax.ShapeDtypeStruct((N, D), jnp.float32),
    )(x, partial)


# speedup vs baseline: 232.1210x; 1.1960x over previous
import jax
import jax.numpy as jnp
from jax import lax
from jax.experimental import pallas as pl
from jax.experimental.pallas import tpu as pltpu
from jax.experimental.pallas import tpu_sc as plsc

N = 10000
E = 320000
D = 128
NPAD = 10240

NC = 2
NS = 16
NW = NC * NS
EPT = 9984
REM = E - NW * EPT
XW = REM // 128
BPT = NPAD // NS


def _sc_hist_body(ei_hbm, out_hbm, iv2, ivx, iv_s, iv_d, ones_v, zero_v,
                  obuf_v, sem, hist_src, hist_dst):
    c = lax.axis_index("c")
    s = lax.axis_index("s")
    w = c * NS + s

    load_m = pltpu.async_copy(ei_hbm.at[:, pl.ds(w * EPT, EPT)], iv2, sem)
    load_x = pltpu.async_copy(
        ei_hbm.at[:, pl.ds(NW * EPT + lax.min(w, XW - 1) * 128, 128)],
        ivx, sem)

    @pl.loop(0, EPT // 16)
    def _(i):
        ones_v[pl.ds(i * 16, 16)] = jnp.ones((16,), jnp.float32)

    @pl.loop(0, BPT // 16)
    def _(i):
        zero_v[pl.ds(i * 16, 16)] = jnp.zeros((16,), jnp.float32)

    pltpu.sync_copy(zero_v, hist_src.at[pl.ds(s * BPT, BPT)])
    pltpu.sync_copy(zero_v, hist_dst.at[pl.ds(s * BPT, BPT)])

    load_m.wait()

    @pl.loop(0, EPT // 16)
    def _(i):
        sl = pl.ds(i * 16, 16)
        iv_s[sl] = iv2[0, sl]
        iv_d[sl] = iv2[1, sl]

    load_x.wait()
    plsc.subcore_barrier()

    pltpu.sync_copy(ones_v, hist_src.at[iv_s], add=True)
    pltpu.sync_copy(ones_v, hist_dst.at[iv_d], add=True)

    @pl.when(w < XW)
    def _():
        @pl.loop(0, 8)
        def _(i):
            sl = pl.ds(i * 16, 16)
            iv_s[sl] = ivx[0, sl]
            iv_d[sl] = ivx[1, sl]

        pltpu.sync_copy(ones_v.at[pl.ds(0, 128)],
                        hist_src.at[iv_s.at[pl.ds(0, 128)]], add=True)
        pltpu.sync_copy(ones_v.at[pl.ds(0, 128)],
                        hist_dst.at[iv_d.at[pl.ds(0, 128)]], add=True)

    plsc.subcore_barrier()

    pltpu.sync_copy(hist_src.at[pl.ds(s * BPT, BPT)], obuf_v)
    pltpu.sync_copy(obuf_v, out_hbm.at[c, 0, pl.ds(s * BPT, BPT)])
    pltpu.sync_copy(hist_dst.at[pl.ds(s * BPT, BPT)], obuf_v)
    pltpu.sync_copy(obuf_v, out_hbm.at[c, 1, pl.ds(s * BPT, BPT)])


_sc_hist = pl.kernel(
    _sc_hist_body,
    out_type=jax.ShapeDtypeStruct((NC, 2, NPAD), jnp.float32),
    mesh=plsc.VectorSubcoreMesh(core_axis_name="c", subcore_axis_name="s",
                                num_cores=NC, num_subcores=NS),
    scratch_types=[
        pltpu.VMEM((2, EPT), jnp.int32),
        pltpu.VMEM((2, 128), jnp.int32),
        pltpu.VMEM((EPT,), jnp.int32),
        pltpu.VMEM((EPT,), jnp.int32),
        pltpu.VMEM((EPT,), jnp.float32),
        pltpu.VMEM((BPT,), jnp.float32),
        pltpu.VMEM((BPT,), jnp.float32),
        pltpu.SemaphoreType.DMA,
        pltpu.VMEM_SHARED((NPAD,), jnp.float32),
        pltpu.VMEM_SHARED((NPAD,), jnp.float32),
    ],
)


def _tc_scale_body(x_ref, p_ref, o_ref):
    p = p_ref[...]
    out_deg = p[0, 0] + p[1, 0]
    in_deg = p[0, 1] + p[1, 1]
    div = jnp.where(in_deg > 0.0, jnp.maximum(out_deg, 1.0), 1.0)
    recip = (1.0 / div)[:N]
    o_ref[...] = x_ref[...] * recip[:, None]


@jax.jit
def kernel(x, edge_index, edge_type, w, b, edge_embeddings):
    del edge_type, w, b, edge_embeddings
    ei = edge_index.astype(jnp.int32)

    partial = _sc_hist(ei)

    return pl.pallas_call(
        _tc_scale_body,
        out_shape=jax.ShapeDtypeStruct((N, D), jnp.float32),
    )(x, partial)
